# baseline (speedup 1.0000x reference)
import functools

import jax
import jax.numpy as jnp
from jax.experimental import pallas as pl
from jax.experimental.pallas import tpu as pltpu

N = 16384
D = 512
H = 256
R = 100
RP = 128
NB = 2048
_NEG = -1e30


def _ln(x, g, b, eps=1e-5):
    n = x.shape[-1]
    s1 = jnp.sum(x, axis=-1, keepdims=True)
    s2 = jnp.sum(x * x, axis=-1, keepdims=True)
    m = s1 * (1.0 / n)
    v = s2 * (1.0 / n) - m * m
    return (x - m) * jax.lax.rsqrt(v + eps) * g + b


def _gelu(x):
    return 0.5 * x * (1.0 + jax.lax.erf(x * 0.7071067811865476))


def _dotT(a, b):
    return jax.lax.dot_general(a, b, (((1,), (1,)), ((), ())),
                               preferred_element_type=jnp.float32)


def _dotC0(a, b):
    return jax.lax.dot_general(a, b, (((0,), (0,)), ((), ())),
                               preferred_element_type=jnp.float32)


def _dot(a, b):
    return jnp.dot(a, b, preferred_element_type=jnp.float32)


def _fused(z_ref, reg_ref, proto_ref, uaW_ref, uab_ref, uag_ref,
           uabe_ref, t_ref, zf_ref, upW_ref, upb_ref, upg_ref, upbe_ref,
           dbW1_ref, dbb1_ref, dbg_ref, dbbe_ref, dbW2_ref, dbb2_ref,
           rr_ref,
           out_ref, zf_res_ref,
           a_scr, s_scr, zfn_scr, bc_scr, z_scr):
    nb = N // NB
    i = pl.program_id(0)
    inv_t = 1.0 / t_ref[0, 0]

    @pl.when(i == 0)
    def _():
        a_scr[...] = jnp.zeros_like(a_scr)
        s_scr[...] = jnp.zeros_like(s_scr)

    @pl.when(i < nb)
    def _phase_a():
        zb = z_ref[...]

        @pl.when(i < nb - 1)
        def _():
            z_scr[pl.dslice(i * NB, NB), :] = zb
        regs = reg_ref[...]

        col = jax.lax.broadcasted_iota(jnp.int32, (NB, RP), 1)
        onehot = (col == regs).astype(jnp.float32)
        valid = col < R

        sim = _dotT(zb, proto_ref[...]) * inv_t
        sim = jnp.where(valid, sim, _NEG)
        m = jnp.max(sim, axis=1, keepdims=True)
        e = jnp.exp(sim - m)
        rowsum = jnp.sum(e, axis=1, keepdims=True)
        own = jnp.sum(e * onehot, axis=1, keepdims=True)
        w_logit = own / rowsum
        e2 = jnp.exp(w_logit)

        ua = _ln(_gelu(_dot(zb, uaW_ref[...]) + uab_ref[...]),
                 uag_ref[...], uabe_ref[...])

        a_scr[...] += _dotC0(onehot, ua * e2)
        s_scr[...] += _dotC0(onehot, e2)

    @pl.when(i == nb)
    def _phase_b():
        s_col = s_scr[...]
        present = s_col > 0.0
        agg = a_scr[...] / jnp.where(present, s_col, 1.0)
        upd = _ln(_gelu(_dot(agg, upW_ref[...]) + upb_ref[...]),
                  upg_ref[...], upbe_ref[...])
        zf = zf_ref[...]
        zf_new = jnp.where(present, upd, zf)
        h = _ln(_gelu(_dot(zf_new, dbW1_ref[...]) + dbb1_ref[...]),
                dbg_ref[...], dbbe_ref[...])
        bc = _dot(h, dbW2_ref[...]) + dbb2_ref[...]
        rw = jax.nn.sigmoid(rr_ref[0, 0])
        zfn_scr[...] = zf_new
        bc_scr[...] = bc
        zf_res_ref[...] = rw * zf_new + (1.0 - rw) * zf

    @pl.when(i >= nb)
    def _phase_c():
        j = i - nb
        zc = z_scr[pl.dslice(jnp.minimum(j, nb - 2) * NB, NB), :]
        z = jnp.where(j == nb - 1, z_ref[...], zc)
        regs = reg_ref[...]
        rw = jax.nn.sigmoid(rr_ref[0, 0])

        col = jax.lax.broadcasted_iota(jnp.int32, (NB, RP), 1)
        onehot = (col == regs).astype(jnp.float32)
        valid = col < R

        sim2 = _dotT(z, zfn_scr[...]) * inv_t
        sim2 = jnp.where(valid, sim2, _NEG)
        m = jnp.max(sim2, axis=1, keepdims=True)
        e = jnp.exp(sim2 - m)
        rowsum = jnp.sum(e, axis=1, keepdims=True)
        own = jnp.sum(e * onehot, axis=1, keepdims=True)
        wt = own / rowsum

        bcg = _dot(onehot, bc_scr[...])
        out_ref[...] = (rw * wt) * bcg + (1.0 - rw) * z


def _full(shape):
    return pl.BlockSpec(shape, lambda *_: tuple(0 for _ in shape))


@jax.jit
def kernel(z_local, z_fused, regions, proto, temperature, raw_rw,
           ua_W, ua_b, ua_g, ua_be, up_W, up_b, up_g, up_be,
           db_W1, db_b1, db_g, db_be, db_W2, db_b2):
    nb = N // NB
    regs2 = regions.reshape(N, 1)
    proto_p = jnp.zeros((RP, D), jnp.float32).at[:R].set(proto)
    zf_p = jnp.zeros((RP, D), jnp.float32).at[:R].set(z_fused)
    t2 = temperature.reshape(1, 1)
    rr2 = raw_rw.reshape(1, 1)

    z_local_res, zf_res = pl.pallas_call(
        _fused,
        grid=(2 * nb,),
        in_specs=[
            pl.BlockSpec((NB, D), lambda i: (jnp.minimum(i, N // NB - 1), 0)),
            pl.BlockSpec((NB, 1), lambda i: (i % (N // NB), 0)),
            _full((RP, D)),
            _full((D, H)),
            _full((1, H)), _full((1, H)), _full((1, H)),
            _full((1, 1)),
            _full((RP, D)),
            _full((H, D)),
            _full((1, D)), _full((1, D)), _full((1, D)),
            _full((D, H)),
            _full((1, H)), _full((1, H)), _full((1, H)),
            _full((H, D)),
            _full((1, D)),
            _full((1, 1)),
        ],
        out_specs=[
            pl.BlockSpec((NB, D), lambda i: (jnp.maximum(i - N // NB, 0), 0)),
            _full((RP, D)),
        ],
        out_shape=[
            jax.ShapeDtypeStruct((N, D), jnp.float32),
            jax.ShapeDtypeStruct((RP, D), jnp.float32),
        ],
        scratch_shapes=[
            pltpu.VMEM((RP, H), jnp.float32),
            pltpu.VMEM((RP, 1), jnp.float32),
            pltpu.VMEM((RP, D), jnp.float32),
            pltpu.VMEM((RP, D), jnp.float32),
            pltpu.VMEM((N - NB, D), jnp.float32),
        ],
    )(z_local, regs2, proto_p, ua_W,
      ua_b.reshape(1, H), ua_g.reshape(1, H), ua_be.reshape(1, H), t2, zf_p,
      up_W, up_b.reshape(1, D), up_g.reshape(1, D), up_be.reshape(1, D),
      db_W1, db_b1.reshape(1, H), db_g.reshape(1, H), db_be.reshape(1, H),
      db_W2, db_b2.reshape(1, D), rr2)

    return (z_local_res, zf_res[:R])


# speedup vs baseline: 1.0134x; 1.0134x over previous
import functools

import jax
import jax.numpy as jnp
from jax.experimental import pallas as pl
from jax.experimental.pallas import tpu as pltpu

N = 16384
D = 512
H = 256
R = 100
RP = 128
NB = 4096
_NEG = -1e30


def _ln(x, g, b, eps=1e-5):
    n = x.shape[-1]
    s1 = jnp.sum(x, axis=-1, keepdims=True)
    s2 = jnp.sum(x * x, axis=-1, keepdims=True)
    m = s1 * (1.0 / n)
    v = s2 * (1.0 / n) - m * m
    return (x - m) * jax.lax.rsqrt(v + eps) * g + b


def _gelu(x):
    return 0.5 * x * (1.0 + jax.lax.erf(x * 0.7071067811865476))


def _dotT(a, b):
    return jax.lax.dot_general(a, b, (((1,), (1,)), ((), ())),
                               preferred_element_type=jnp.float32)


def _dotC0(a, b):
    return jax.lax.dot_general(a, b, (((0,), (0,)), ((), ())),
                               preferred_element_type=jnp.float32)


def _dot(a, b):
    return jnp.dot(a, b, preferred_element_type=jnp.float32)


def _fused(z_ref, reg_ref, proto_ref, uaW_ref, uab_ref, uag_ref,
           uabe_ref, t_ref, zf_ref, upW_ref, upb_ref, upg_ref, upbe_ref,
           dbW1_ref, dbb1_ref, dbg_ref, dbbe_ref, dbW2_ref, dbb2_ref,
           rr_ref,
           out_ref, zf_res_ref,
           a_scr, s_scr, zfn_scr, bc_scr):
    nb = N // NB
    i = pl.program_id(0)
    inv_t = 1.0 / t_ref[0, 0]

    @pl.when(i == 0)
    def _():
        a_scr[...] = jnp.zeros_like(a_scr)
        s_scr[...] = jnp.zeros_like(s_scr)

    @pl.when(i < nb)
    def _phase_a():
        zb = z_ref[...]
        regs = reg_ref[...]

        col = jax.lax.broadcasted_iota(jnp.int32, (NB, RP), 1)
        onehot = (col == regs).astype(jnp.float32)
        valid = col < R

        sim = _dotT(zb, proto_ref[...]) * inv_t
        sim = jnp.where(valid, sim, _NEG)
        m = jnp.max(sim, axis=1, keepdims=True)
        e = jnp.exp(sim - m)
        rowsum = jnp.sum(e, axis=1, keepdims=True)
        own = jnp.sum(e * onehot, axis=1, keepdims=True)
        w_logit = own / rowsum
        e2 = jnp.exp(w_logit)

        ua = _ln(_gelu(_dot(zb, uaW_ref[...]) + uab_ref[...]),
                 uag_ref[...], uabe_ref[...])

        a_scr[...] += _dotC0(onehot, ua * e2)
        s_scr[...] += _dotC0(onehot, e2)

    @pl.when(i == nb)
    def _phase_b():
        s_col = s_scr[...]
        present = s_col > 0.0
        agg = a_scr[...] / jnp.where(present, s_col, 1.0)
        upd = _ln(_gelu(_dot(agg, upW_ref[...]) + upb_ref[...]),
                  upg_ref[...], upbe_ref[...])
        zf = zf_ref[...]
        zf_new = jnp.where(present, upd, zf)
        h = _ln(_gelu(_dot(zf_new, dbW1_ref[...]) + dbb1_ref[...]),
                dbg_ref[...], dbbe_ref[...])
        bc = _dot(h, dbW2_ref[...]) + dbb2_ref[...]
        rw = jax.nn.sigmoid(rr_ref[0, 0])
        zfn_scr[...] = zf_new
        bc_scr[...] = bc
        zf_res_ref[...] = rw * zf_new + (1.0 - rw) * zf

    @pl.when(i >= nb)
    def _phase_c():
        z = z_ref[...]
        regs = reg_ref[...]
        rw = jax.nn.sigmoid(rr_ref[0, 0])

        col = jax.lax.broadcasted_iota(jnp.int32, (NB, RP), 1)
        onehot = (col == regs).astype(jnp.float32)
        valid = col < R

        sim2 = _dotT(z, zfn_scr[...]) * inv_t
        sim2 = jnp.where(valid, sim2, _NEG)
        m = jnp.max(sim2, axis=1, keepdims=True)
        e = jnp.exp(sim2 - m)
        rowsum = jnp.sum(e, axis=1, keepdims=True)
        own = jnp.sum(e * onehot, axis=1, keepdims=True)
        wt = own / rowsum

        bcg = _dot(onehot, bc_scr[...])
        out_ref[...] = (rw * wt) * bcg + (1.0 - rw) * z


def _full(shape):
    return pl.BlockSpec(shape, lambda *_: tuple(0 for _ in shape))


@jax.jit
def kernel(z_local, z_fused, regions, proto, temperature, raw_rw,
           ua_W, ua_b, ua_g, ua_be, up_W, up_b, up_g, up_be,
           db_W1, db_b1, db_g, db_be, db_W2, db_b2):
    nb = N // NB
    regs2 = regions.reshape(N, 1)
    proto_p = jnp.zeros((RP, D), jnp.float32).at[:R].set(proto)
    zf_p = jnp.zeros((RP, D), jnp.float32).at[:R].set(z_fused)
    t2 = temperature.reshape(1, 1)
    rr2 = raw_rw.reshape(1, 1)

    z_local_res, zf_res = pl.pallas_call(
        _fused,
        grid=(2 * nb,),
        in_specs=[
            pl.BlockSpec((NB, D), lambda i: (i % (N // NB), 0)),
            pl.BlockSpec((NB, 1), lambda i: (i % (N // NB), 0)),
            _full((RP, D)),
            _full((D, H)),
            _full((1, H)), _full((1, H)), _full((1, H)),
            _full((1, 1)),
            _full((RP, D)),
            _full((H, D)),
            _full((1, D)), _full((1, D)), _full((1, D)),
            _full((D, H)),
            _full((1, H)), _full((1, H)), _full((1, H)),
            _full((H, D)),
            _full((1, D)),
            _full((1, 1)),
        ],
        out_specs=[
            pl.BlockSpec((NB, D), lambda i: (jnp.maximum(i - N // NB, 0), 0)),
            _full((RP, D)),
        ],
        out_shape=[
            jax.ShapeDtypeStruct((N, D), jnp.float32),
            jax.ShapeDtypeStruct((RP, D), jnp.float32),
        ],
        scratch_shapes=[
            pltpu.VMEM((RP, H), jnp.float32),
            pltpu.VMEM((RP, 1), jnp.float32),
            pltpu.VMEM((RP, D), jnp.float32),
            pltpu.VMEM((RP, D), jnp.float32),
        ],
    )(z_local, regs2, proto_p, ua_W,
      ua_b.reshape(1, H), ua_g.reshape(1, H), ua_be.reshape(1, H), t2, zf_p,
      up_W, up_b.reshape(1, D), up_g.reshape(1, D), up_be.reshape(1, D),
      db_W1, db_b1.reshape(1, H), db_g.reshape(1, H), db_be.reshape(1, H),
      db_W2, db_b2.reshape(1, D), rr2)

    return (z_local_res, zf_res[:R])
